# Initial kernel scaffold; baseline (speedup 1.0000x reference)
import jax
import jax.numpy as jnp
from jax.experimental import pallas as pl


def kernel(x):
    raise NotImplementedError("write your pallas kernel here")


# speedup vs baseline: 25.0540x; 25.0540x over previous
import functools

import jax
import jax.numpy as jnp
from jax.experimental import pallas as pl

_BLOCK_ROWS = 512


def _rowmask_kernel(x_ref, o_ref):
    x = x_ref[...]
    rows, cols = x.shape
    rowmax = jnp.max(x, axis=1, keepdims=True)
    col = jax.lax.broadcasted_iota(jnp.int32, (rows, cols), 1)
    amax = jnp.min(jnp.where(x == rowmax, col, cols), axis=1, keepdims=True)
    o_ref[...] = jnp.where(col == amax, x, 0.0)


@functools.partial(jax.jit, static_argnames=())
def kernel(x):
    n, d = x.shape
    grid = (n // _BLOCK_ROWS,)
    return pl.pallas_call(
        _rowmask_kernel,
        grid=grid,
        in_specs=[pl.BlockSpec((_BLOCK_ROWS, d), lambda i: (i, 0))],
        out_specs=pl.BlockSpec((_BLOCK_ROWS, d), lambda i: (i, 0)),
        out_shape=jax.ShapeDtypeStruct((n, d), jnp.float32),
    )(x)


# speedup vs baseline: 25.6217x; 1.0227x over previous
import functools

import jax
import jax.numpy as jnp
from jax.experimental import pallas as pl

_BLOCK_ROWS = 1024


def _rowmask_kernel(x_ref, o_ref):
    x = x_ref[...]
    rows, cols = x.shape
    rowmax = jnp.max(x, axis=1, keepdims=True)
    col = jax.lax.broadcasted_iota(jnp.int32, (rows, cols), 1)
    amax = jnp.min(jnp.where(x == rowmax, col, cols), axis=1, keepdims=True)
    o_ref[...] = jnp.where(col == amax, x, 0.0)


@functools.partial(jax.jit, static_argnames=())
def kernel(x):
    n, d = x.shape
    grid = (n // _BLOCK_ROWS,)
    return pl.pallas_call(
        _rowmask_kernel,
        grid=grid,
        in_specs=[pl.BlockSpec((_BLOCK_ROWS, d), lambda i: (i, 0))],
        out_specs=pl.BlockSpec((_BLOCK_ROWS, d), lambda i: (i, 0)),
        out_shape=jax.ShapeDtypeStruct((n, d), jnp.float32),
    )(x)
